# Initial kernel scaffold; baseline (speedup 1.0000x reference)
import jax
import jax.numpy as jnp
from jax.experimental import pallas as pl


def kernel(feature, batch, a):
    raise NotImplementedError("write your pallas kernel here")


# speedup vs baseline: 12.5125x; 12.5125x over previous
import functools

import jax
import jax.numpy as jnp
from jax import lax
from jax.experimental import pallas as pl
from jax.experimental.pallas import tpu as pltpu
from jax.experimental.pallas import tpu_sc as plsc

N = 100000
D = 128
S = 512
W = 32
PW = 3200
NP = W * PW
SEG_PAD = 640
BLK = 256
SIZES = [BLK] * 12 + [128]
NBLK = len(SIZES)
C16 = D // 16
NEG_INF = -3.4e38

_mesh = plsc.VectorSubcoreMesh(core_axis_name="c", subcore_axis_name="s")


@functools.partial(
    pl.kernel,
    out_type=(jax.ShapeDtypeStruct((NP,), jnp.float32),
              jax.ShapeDtypeStruct((W, 16), jnp.float32)),
    mesh=_mesh,
    compiler_params=pltpu.CompilerParams(needs_layout_passes=False),
    scratch_types=[
        pltpu.VMEM((2, BLK, D), jnp.float32),
        pltpu.VMEM((D,), jnp.float32),
        pltpu.VMEM((PW,), jnp.float32),
        pltpu.VMEM((16,), jnp.float32),
        pltpu.SemaphoreType.DMA,
        pltpu.SemaphoreType.DMA,
    ],
)
def _logits_kernel(feat_hbm, a_hbm, logits_hbm, gmax_hbm,
                   fbuf, av, lbuf, gbuf, sem0, sem1):
    cid = lax.axis_index("c")
    sid = lax.axis_index("s")
    wid = sid * 2 + cid
    row0 = wid * PW
    pltpu.sync_copy(a_hbm, av)
    a_chunks = [av[pl.ds(16 * c, 16)] for c in range(C16)]
    sems = (sem0, sem1)

    def start(j):
        sz = SIZES[j]
        return pltpu.async_copy(
            feat_hbm.at[pl.ds(row0 + j * BLK, sz), :],
            fbuf.at[j % 2].at[pl.ds(0, sz)], sems[j % 2])

    cps = [start(0), None]
    lane = lax.iota(jnp.int32, 16)
    masks = [lane == i for i in range(16)]
    m16 = jnp.full((16,), jnp.float32(NEG_INF))
    for j in range(NBLK):
        cps[j % 2].wait()
        if j + 1 < NBLK:
            cps[(j + 1) % 2] = start(j + 1)
        base = j * BLK
        bufi = j % 2

        def grp_body(g, mm, _base=base, _bufi=bufi):
            l16 = jnp.zeros((16,), jnp.float32)
            for i in range(16):
                r = 16 * g + i
                p = fbuf[_bufi, r, pl.ds(0, 16)] * a_chunks[0]
                for c in range(1, C16):
                    p = p + fbuf[_bufi, r, pl.ds(16 * c, 16)] * a_chunks[c]
                s_ = jnp.sum(p)
                l_ = jnp.where(s_ > 0, s_, jnp.float32(0.2) * s_)
                l16 = jnp.where(masks[i], l_, l16)
            lbuf[pl.ds(_base + 16 * g, 16)] = l16
            return jnp.maximum(mm, l16)

        m16 = lax.fori_loop(0, SIZES[j] // 16, grp_body, m16)
    pltpu.sync_copy(lbuf, logits_hbm.at[pl.ds(row0, PW)])
    gbuf[...] = m16
    pltpu.sync_copy(gbuf, gmax_hbm.at[wid])


@functools.partial(
    pl.kernel,
    out_type=(jax.ShapeDtypeStruct((2, SEG_PAD, D), jnp.float32),
              jax.ShapeDtypeStruct((2, SEG_PAD), jnp.float32)),
    mesh=_mesh,
    compiler_params=pltpu.CompilerParams(needs_layout_passes=False),
    scratch_types=[
        pltpu.VMEM((2, BLK, D), jnp.float32),
        pltpu.VMEM((32 + PW,), jnp.int32),
        pltpu.VMEM((PW,), jnp.float32),
        pltpu.VMEM((16 + PW,), jnp.float32),
        pltpu.VMEM((3232,), jnp.int32),
        pltpu.VMEM((W, 16), jnp.float32),
        pltpu.VMEM((40, D), jnp.float32),
        pltpu.VMEM((SEG_PAD,), jnp.float32),
        pltpu.VMEM((16, D), jnp.float32),
        pltpu.VMEM((16,), jnp.int32),
        pltpu.VMEM((16,), jnp.float32),
        pltpu.VMEM_SHARED((SEG_PAD, D), jnp.float32),
        pltpu.VMEM_SHARED((SEG_PAD,), jnp.float32),
        pltpu.SemaphoreType.DMA,
        pltpu.SemaphoreType.DMA,
    ],
)
def _pool_kernel(feat_hbm, batch_hbm, logits_hbm, gmax_hbm, u_hbm, dpart_hbm,
                 fbuf, bbuf, lbuf, att, bnd, gv, zrows, zd, flushb, idxb,
                 dbuf, u_sh, d_sh, sem0, sem1):
    cid = lax.axis_index("c")
    sid = lax.axis_index("s")
    wid = sid * 2 + cid
    row0 = wid * PW
    sems = (sem0, sem1)
    lane = lax.iota(jnp.int32, 16)
    lane0 = lane == 0
    dead = jnp.int32(SEG_PAD - 1)

    bbuf[pl.ds(0, 16)] = jnp.full((16,), jnp.int32(-1))
    pltpu.sync_copy(batch_hbm.at[pl.ds(row0, PW)], bbuf.at[pl.ds(16, PW)])
    pltpu.sync_copy(logits_hbm.at[pl.ds(row0, PW)], lbuf)
    pltpu.sync_copy(gmax_hbm, gv)

    def gbody(i, acc):
        return jnp.maximum(acc, gv[i, :])
    g16 = lax.fori_loop(1, W, gbody, gv[0, :])
    G = jnp.max(g16)

    def abody(i, cnt):
        l16 = lbuf[pl.ds(16 * i, 16)]
        att[pl.ds(16 * i, 16)] = jnp.exp(l16 - G)
        b16 = bbuf[pl.ds(16 + 16 * i, 16)]
        p16 = bbuf[pl.ds(15 + 16 * i, 16)]
        neq = b16 != p16
        idx16 = lane + 16 * i
        plsc.store_compressed(bnd.at[pl.ds(cnt, 16)], idx16, mask=neq)
        return cnt + jnp.max(plsc.all_reduce_population_count(neq))

    cnt = lax.fori_loop(0, PW // 16, abody, jnp.int32(0))
    bnd[pl.ds(cnt, 16)] = jnp.full((16,), jnp.int32(PW))

    z16 = jnp.zeros((16,), jnp.float32)
    for r in range(40):
        for c in range(C16):
            zrows[r, pl.ds(16 * c, 16)] = z16
    for r in range(SEG_PAD // 16):
        zd[pl.ds(16 * r, 16)] = z16
    for r in range(16):
        for c in range(C16):
            flushb[r, pl.ds(16 * c, 16)] = z16
    pltpu.sync_copy(zrows, u_sh.at[pl.ds(sid * 40, 40), :])

    @pl.when(sid == 0)
    def _():
        pltpu.sync_copy(zd, d_sh)

    plsc.subcore_barrier()

    def start(j):
        sz = SIZES[j]
        return pltpu.async_copy(
            feat_hbm.at[pl.ds(row0 + j * BLK, sz), :],
            fbuf.at[j % 2].at[pl.ds(0, sz)], sems[j % 2])

    cps = [start(0), None]
    carry = (jnp.int32(0), jnp.int32(0), jnp.float32(0)) + \
        tuple(z16 for _ in range(C16))
    for j in range(NBLK):
        cps[j % 2].wait()
        if j + 1 < NBLK:
            cps[(j + 1) % 2] = start(j + 1)
        B0 = j * BLK
        B1 = B0 + SIZES[j]
        bufi = j % 2

        def wcond(st, _B1=B1):
            return st[1] < _B1

        def wbody(st, _B0=B0, _B1=B1, _bufi=bufi):
            k, pos, dsum, *acc = st
            nb = bnd[pl.ds(k + 1, 16)][0]
            e = jnp.minimum(nb, _B1)
            seg = bbuf[pl.ds(16 + pos, 16)][0]

            def ibody(i, c2, _pos=pos):
                ds_, *ac = c2
                r = _pos + i
                a_s = att[pl.ds(r, 16)][0]
                rb = r - _B0
                nac = [ac[c] + a_s * fbuf[_bufi, rb, pl.ds(16 * c, 16)]
                       for c in range(C16)]
                return (ds_ + a_s, *nac)

            dsum, *acc = lax.fori_loop(0, e - pos, ibody, (dsum, *acc))
            done = nb <= _B1

            @pl.when(done)
            def _():
                for c in range(C16):
                    flushb[0, pl.ds(16 * c, 16)] = acc[c]
                idxb[...] = jnp.where(lane0, seg, dead)
                dbuf[...] = jnp.where(lane0, dsum, jnp.float32(0))
                pltpu.sync_copy(flushb, u_sh.at[idxb], add=True)
                pltpu.sync_copy(dbuf, d_sh.at[idxb], add=True)

            keep = jnp.where(done, jnp.float32(0), jnp.float32(1))
            acc = [a * keep for a in acc]
            dsum = dsum * keep
            k = jnp.where(done, k + 1, k)
            return (k, e, dsum, *acc)

        carry = lax.while_loop(wcond, wbody, carry)

    plsc.subcore_barrier()
    pltpu.sync_copy(u_sh.at[pl.ds(sid * 40, 40), :],
                    u_hbm.at[cid].at[pl.ds(sid * 40, 40), :])

    @pl.when(sid == 0)
    def _():
        pltpu.sync_copy(d_sh, dpart_hbm.at[cid])


@functools.partial(
    pl.kernel,
    out_type=jax.ShapeDtypeStruct((S, D), jnp.float32),
    mesh=_mesh,
    compiler_params=pltpu.CompilerParams(needs_layout_passes=False),
    scratch_types=[
        pltpu.VMEM((2, 16, D), jnp.float32),
        pltpu.VMEM((2, SEG_PAD), jnp.float32),
        pltpu.VMEM((16, D), jnp.float32),
    ],
)
def _combine_kernel(u_hbm, d_hbm, out_hbm, ub, db, ob):
    cid = lax.axis_index("c")
    sid = lax.axis_index("s")
    wid = sid * 2 + cid
    r0 = wid * 16
    pltpu.sync_copy(u_hbm.at[0].at[pl.ds(r0, 16), :], ub.at[0])
    pltpu.sync_copy(u_hbm.at[1].at[pl.ds(r0, 16), :], ub.at[1])
    pltpu.sync_copy(d_hbm, db)
    d16 = db[0, pl.ds(r0, 16)] + db[1, pl.ds(r0, 16)]
    inv16 = jnp.where(d16 > 0, jnp.float32(1) / d16, jnp.float32(0))
    for r in range(16):
        ivr = inv16[r]
        for c in range(C16):
            ob[r, pl.ds(16 * c, 16)] = (
                ub[0, r, pl.ds(16 * c, 16)] + ub[1, r, pl.ds(16 * c, 16)]
            ) * ivr
    pltpu.sync_copy(ob, out_hbm.at[pl.ds(r0, 16), :])


def kernel(feature, batch, a):
    feat_p = jnp.concatenate(
        [feature, jnp.zeros((NP - N, D), feature.dtype)], axis=0)
    batch_p = jnp.concatenate(
        [batch, jnp.full((NP - N,), S, batch.dtype)], axis=0)
    a_v = a.reshape(D).astype(jnp.float32)
    logits, gmax = _logits_kernel(feat_p, a_v)
    u, dpart = _pool_kernel(feat_p, batch_p, logits, gmax)
    return _combine_kernel(u, dpart)


# speedup vs baseline: 13.2097x; 1.0557x over previous
import functools

import jax
import jax.numpy as jnp
from jax import lax
from jax.experimental import pallas as pl
from jax.experimental.pallas import tpu as pltpu
from jax.experimental.pallas import tpu_sc as plsc

N = 100000
D = 128
S = 512
W = 32
PW = 3200
NP = W * PW
SEG_PAD = 640
BLK = 256
SIZES = [BLK] * 12 + [128]
NBLK = len(SIZES)
C16 = D // 16
NEG_INF = -3.4e38

_mesh = plsc.VectorSubcoreMesh(core_axis_name="c", subcore_axis_name="s")


@functools.partial(
    pl.kernel,
    out_type=(jax.ShapeDtypeStruct((2, SEG_PAD, D), jnp.float32),
              jax.ShapeDtypeStruct((2, SEG_PAD), jnp.float32),
              jax.ShapeDtypeStruct((2, 16), jnp.float32)),
    mesh=_mesh,
    compiler_params=pltpu.CompilerParams(needs_layout_passes=False),
    scratch_types=[
        pltpu.VMEM((2, BLK, D), jnp.float32),
        pltpu.VMEM((D,), jnp.float32),
        pltpu.VMEM((PW,), jnp.float32),
        pltpu.VMEM((32 + PW,), jnp.int32),
        pltpu.VMEM((16 + PW,), jnp.float32),
        pltpu.VMEM((3232,), jnp.int32),
        pltpu.VMEM((16,), jnp.float32),
        pltpu.VMEM((16,), jnp.int32),
        pltpu.VMEM((256,), jnp.float32),
        pltpu.VMEM((40, D), jnp.float32),
        pltpu.VMEM((SEG_PAD,), jnp.float32),
        pltpu.VMEM((16, D), jnp.float32),
        pltpu.VMEM((16,), jnp.int32),
        pltpu.VMEM((16,), jnp.float32),
        pltpu.VMEM_SHARED((SEG_PAD, D), jnp.float32),
        pltpu.VMEM_SHARED((SEG_PAD,), jnp.float32),
        pltpu.VMEM_SHARED((256,), jnp.float32),
        pltpu.SemaphoreType.DMA,
        pltpu.SemaphoreType.DMA,
    ],
)
def _pool_kernel(feat_hbm, batch_hbm, a_hbm, u_hbm, dpart_hbm, gs_hbm,
                 fbuf, av, lbuf, bbuf, att, bnd, gbuf, gidx, mbuf,
                 zrows, zd, flushb, idxb, dbuf, u_sh, d_sh, m_sh,
                 sem0, sem1):
    cid = lax.axis_index("c")
    sid = lax.axis_index("s")
    wid = sid * 2 + cid
    row0 = wid * PW
    sems = (sem0, sem1)
    lane = lax.iota(jnp.int32, 16)
    lane0 = lane == 0
    masks = [lane == i for i in range(16)]
    dead = jnp.int32(SEG_PAD - 1)

    def start(j):
        sz = SIZES[j]
        return pltpu.async_copy(
            feat_hbm.at[pl.ds(row0 + j * BLK, sz), :],
            fbuf.at[j % 2].at[pl.ds(0, sz)], sems[j % 2])

    cps = [start(0), None]
    pltpu.sync_copy(a_hbm, av)
    a_chunks = [av[pl.ds(16 * c, 16)] for c in range(C16)]
    bbuf[pl.ds(0, 16)] = jnp.full((16,), jnp.int32(-1))
    pltpu.sync_copy(batch_hbm.at[pl.ds(row0, PW)], bbuf.at[pl.ds(16, PW)])

    m16 = jnp.full((16,), jnp.float32(NEG_INF))
    for j in range(NBLK):
        cps[j % 2].wait()
        if j + 1 < NBLK:
            cps[(j + 1) % 2] = start(j + 1)
        base = j * BLK
        bufi = j % 2

        def grp_body(g, mm, _base=base, _bufi=bufi):
            l16 = jnp.zeros((16,), jnp.float32)
            for i in range(16):
                r = 16 * g + i
                p = fbuf[_bufi, r, pl.ds(0, 16)] * a_chunks[0]
                for c in range(1, C16):
                    p = p + fbuf[_bufi, r, pl.ds(16 * c, 16)] * a_chunks[c]
                s_ = jnp.sum(p)
                l_ = jnp.where(s_ > 0, s_, jnp.float32(0.2) * s_)
                l16 = jnp.where(masks[i], l_, l16)
            lbuf[pl.ds(_base + 16 * g, 16)] = l16
            return jnp.maximum(mm, l16)

        m16 = lax.fori_loop(0, SIZES[j] // 16, grp_body, m16)

    gbuf[...] = m16
    gidx[...] = sid * 16 + lane
    pltpu.sync_copy(gbuf, m_sh.at[gidx])
    z16 = jnp.zeros((16,), jnp.float32)
    for r in range(40):
        for c in range(C16):
            zrows[r, pl.ds(16 * c, 16)] = z16
    for r in range(SEG_PAD // 16):
        zd[pl.ds(16 * r, 16)] = z16
    for r in range(16):
        for c in range(C16):
            flushb[r, pl.ds(16 * c, 16)] = z16
    pltpu.sync_copy(zrows, u_sh.at[pl.ds(sid * 40, 40), :])

    @pl.when(sid == 0)
    def _():
        pltpu.sync_copy(zd, d_sh)

    plsc.subcore_barrier()

    cps = [start(0), None]
    pltpu.sync_copy(m_sh, mbuf)

    def mbody(i, acc):
        return jnp.maximum(acc, mbuf[pl.ds(16 * i, 16)])
    g16 = lax.fori_loop(1, 16, mbody, mbuf[pl.ds(0, 16)])
    G = jnp.max(g16)

    def abody(i, cnt):
        l16 = lbuf[pl.ds(16 * i, 16)]
        att[pl.ds(16 * i, 16)] = jnp.exp(l16 - G)
        b16 = bbuf[pl.ds(16 + 16 * i, 16)]
        p16 = bbuf[pl.ds(15 + 16 * i, 16)]
        neq = b16 != p16
        idx16 = lane + 16 * i
        plsc.store_compressed(bnd.at[pl.ds(cnt, 16)], idx16, mask=neq)
        return cnt + jnp.max(plsc.all_reduce_population_count(neq))

    cnt = lax.fori_loop(0, PW // 16, abody, jnp.int32(0))
    bnd[pl.ds(cnt, 16)] = jnp.full((16,), jnp.int32(PW))

    carry = (jnp.int32(0), jnp.int32(0), jnp.float32(0)) + \
        tuple(z16 for _ in range(C16))
    for j in range(NBLK):
        cps[j % 2].wait()
        if j + 1 < NBLK:
            cps[(j + 1) % 2] = start(j + 1)
        B0 = j * BLK
        B1 = B0 + SIZES[j]
        bufi = j % 2

        def wcond(st, _B1=B1):
            return st[1] < _B1

        def wbody(st, _B0=B0, _B1=B1, _bufi=bufi):
            k, pos, dsum, *acc = st
            nb = bnd[pl.ds(k + 1, 16)][0]
            e = jnp.minimum(nb, _B1)
            seg = bbuf[pl.ds(16 + pos, 16)][0]

            def ibody(i, c2, _pos=pos):
                ds_, *ac = c2
                r = _pos + i
                a_s = att[pl.ds(r, 16)][0]
                rb = r - _B0
                nac = [ac[c] + a_s * fbuf[_bufi, rb, pl.ds(16 * c, 16)]
                       for c in range(C16)]
                return (ds_ + a_s, *nac)

            dsum, *acc = lax.fori_loop(0, e - pos, ibody, (dsum, *acc))
            done = nb <= _B1

            @pl.when(done)
            def _():
                for c in range(C16):
                    flushb[0, pl.ds(16 * c, 16)] = acc[c]
                idxb[...] = jnp.where(lane0, seg, dead)
                dbuf[...] = jnp.where(lane0, dsum, jnp.float32(0))
                pltpu.sync_copy(flushb, u_sh.at[idxb], add=True)
                pltpu.sync_copy(dbuf, d_sh.at[idxb], add=True)

            keep = jnp.where(done, jnp.float32(0), jnp.float32(1))
            acc = [a * keep for a in acc]
            dsum = dsum * keep
            k = jnp.where(done, k + 1, k)
            return (k, e, dsum, *acc)

        carry = lax.while_loop(wcond, wbody, carry)

    plsc.subcore_barrier()
    pltpu.sync_copy(u_sh.at[pl.ds(sid * 40, 40), :],
                    u_hbm.at[cid].at[pl.ds(sid * 40, 40), :])

    @pl.when(sid == 0)
    def _():
        pltpu.sync_copy(d_sh, dpart_hbm.at[cid])
        gbuf[...] = jnp.zeros((16,), jnp.float32) + G
        pltpu.sync_copy(gbuf, gs_hbm.at[cid])


@functools.partial(
    pl.kernel,
    out_type=jax.ShapeDtypeStruct((S, D), jnp.float32),
    mesh=_mesh,
    compiler_params=pltpu.CompilerParams(needs_layout_passes=False),
    scratch_types=[
        pltpu.VMEM((2, 16, D), jnp.float32),
        pltpu.VMEM((2, SEG_PAD), jnp.float32),
        pltpu.VMEM((2, 16), jnp.float32),
        pltpu.VMEM((16, D), jnp.float32),
    ],
)
def _combine_kernel(u_hbm, d_hbm, gs_hbm, out_hbm, ub, db, gsb, ob):
    cid = lax.axis_index("c")
    sid = lax.axis_index("s")
    wid = sid * 2 + cid
    r0 = wid * 16
    pltpu.sync_copy(u_hbm.at[0].at[pl.ds(r0, 16), :], ub.at[0])
    pltpu.sync_copy(u_hbm.at[1].at[pl.ds(r0, 16), :], ub.at[1])
    pltpu.sync_copy(d_hbm, db)
    pltpu.sync_copy(gs_hbm, gsb)
    g0v = gsb[0, :]
    g1v = gsb[1, :]
    gm = jnp.maximum(jnp.max(g0v), jnp.max(g1v))
    s0 = jnp.exp(g0v - gm)[0]
    s1 = jnp.exp(g1v - gm)[0]
    d16 = db[0, pl.ds(r0, 16)] * s0 + db[1, pl.ds(r0, 16)] * s1
    inv16 = jnp.where(d16 > 0, jnp.float32(1) / d16, jnp.float32(0))
    for r in range(16):
        ivr = inv16[r]
        for c in range(C16):
            ob[r, pl.ds(16 * c, 16)] = (
                ub[0, r, pl.ds(16 * c, 16)] * s0
                + ub[1, r, pl.ds(16 * c, 16)] * s1
            ) * ivr
    pltpu.sync_copy(ob, out_hbm.at[pl.ds(r0, 16), :])


def kernel(feature, batch, a):
    feat_p = jnp.concatenate(
        [feature, jnp.zeros((NP - N, D), feature.dtype)], axis=0)
    batch_p = jnp.concatenate(
        [batch, jnp.full((NP - N,), S, batch.dtype)], axis=0)
    a_v = a.reshape(D).astype(jnp.float32)
    u, dpart, gs = _pool_kernel(feat_p, batch_p, a_v)
    return _combine_kernel(u, dpart, gs)
